# Initial kernel scaffold; baseline (speedup 1.0000x reference)
import jax
import jax.numpy as jnp
from jax.experimental import pallas as pl


def kernel(x, router, router_bias, w_gate_up, w_down):
    raise NotImplementedError("write your pallas kernel here")


# speedup vs baseline: 1.8143x; 1.8143x over previous
import functools

import jax
import jax.numpy as jnp
from jax.experimental import pallas as pl
from jax.experimental.pallas import tpu as pltpu

T, D, I, E, K = 2048, 1024, 512, 16, 2
NEG_INF = -1e30


def _router_kernel(x_ref, w_ref, b_ref, c_ref, counts_ref, ent_ref):
    x = x_ref[...]
    w = w_ref[...]
    logits = jnp.dot(x, w, preferred_element_type=jnp.float32)
    biased = logits + b_ref[...]

    iota_e = jax.lax.broadcasted_iota(jnp.int32, (T, E), 1)
    m0 = jnp.max(biased, axis=-1, keepdims=True)
    e0 = jnp.min(jnp.where(biased == m0, iota_e, E), axis=-1, keepdims=True)
    masked = jnp.where(iota_e == e0, NEG_INF, biased)
    m1 = jnp.max(masked, axis=-1, keepdims=True)
    e1 = jnp.min(jnp.where(masked == m1, iota_e, E), axis=-1, keepdims=True)

    one0 = (iota_e == e0).astype(jnp.float32)
    one1 = (iota_e == e1).astype(jnp.float32)
    u0 = jnp.sum(jnp.where(iota_e == e0, logits, 0.0), axis=-1, keepdims=True)
    u1 = jnp.sum(jnp.where(iota_e == e1, logits, 0.0), axis=-1, keepdims=True)
    s0 = jax.nn.sigmoid(u0)
    s1 = jax.nn.sigmoid(u1)
    denom = s0 + s1
    w0 = s0 / denom
    w1 = s1 / denom
    c_ref[...] = one0 * w0 + one1 * w1

    counts = jnp.sum(one0 + one1, axis=0, keepdims=True)
    counts_ref[...] = counts
    total = jnp.maximum(jnp.sum(counts), 1.0)
    frac = counts / total
    ent_ref[...] = -jnp.sum(frac * jnp.log(frac + 1e-6), keepdims=True).reshape(1, 1)


def _moe_dense_kernel(x_ref, c_ref, wgu_ref, wd_ref, out_ref):
    e = pl.program_id(1)

    @pl.when(e == 0)
    def _():
        out_ref[...] = jnp.zeros_like(out_ref)

    x = x_ref[...]
    gu = jnp.dot(x, wgu_ref[0], preferred_element_type=jnp.float32)
    gate = gu[:, :I]
    up = gu[:, I:]
    h = gate * jax.nn.sigmoid(gate) * up
    y = jnp.dot(h, wd_ref[0], preferred_element_type=jnp.float32)
    iota_e = jax.lax.broadcasted_iota(jnp.int32, (out_ref.shape[0], E), 1)
    c = jnp.sum(jnp.where(iota_e == e, c_ref[...], 0.0), axis=1, keepdims=True)
    out_ref[...] += c * y


@jax.jit
def kernel(x, router, router_bias, w_gate_up, w_down):
    x_flat = x.reshape(T, D)

    c, counts, ent = pl.pallas_call(
        _router_kernel,
        out_shape=[
            jax.ShapeDtypeStruct((T, E), jnp.float32),
            jax.ShapeDtypeStruct((1, E), jnp.float32),
            jax.ShapeDtypeStruct((1, 1), jnp.float32),
        ],
    )(x_flat, router, router_bias.reshape(1, E))

    BT = 512
    routed = pl.pallas_call(
        _moe_dense_kernel,
        grid=(T // BT, E),
        in_specs=[
            pl.BlockSpec((BT, D), lambda t, e: (t, 0)),
            pl.BlockSpec((BT, E), lambda t, e: (t, 0)),
            pl.BlockSpec((1, D, 2 * I), lambda t, e: (e, 0, 0)),
            pl.BlockSpec((1, I, D), lambda t, e: (e, 0, 0)),
        ],
        out_specs=pl.BlockSpec((BT, D), lambda t, e: (t, 0)),
        out_shape=jax.ShapeDtypeStruct((T, D), jnp.float32),
        compiler_params=pltpu.CompilerParams(
            dimension_semantics=("parallel", "arbitrary"),
        ),
    )(x_flat, c, w_gate_up, w_down)

    return routed.reshape(x.shape), counts.reshape(E), ent.reshape(())
